# baseline (speedup 1.0000x reference)
import functools

import jax
import jax.numpy as jnp
from jax import lax
from jax.experimental import pallas as pl
from jax.experimental.pallas import tpu as pltpu
from jax.experimental.pallas import tpu_sc as plsc

_NUM_CORES = 2
_NUM_SUBCORES = 16
_NUM_WORKERS = _NUM_CORES * _NUM_SUBCORES

_R = 4
_NB = 3
_SPLIT = 128


def _gather_call(x, w):
    n_rows, seq = x.shape
    vocab, d = w.shape
    rows_per_w = n_rows // _NUM_WORKERS
    n_chunks = rows_per_w // _R
    splits = [(0, _SPLIT), (_SPLIT, seq - _SPLIT)]

    mesh = plsc.VectorSubcoreMesh(
        core_axis_name="c", subcore_axis_name="s")

    @functools.partial(
        pl.kernel,
        mesh=mesh,
        compiler_params=pltpu.CompilerParams(use_tc_tiling_on_sc=False),
        out_type=jax.ShapeDtypeStruct((n_rows, seq, d), jnp.float32),
        scratch_types=[
            pltpu.VMEM((rows_per_w, seq), jnp.int32),
            pltpu.VMEM((_NB, _R, seq, d), jnp.float32),
            [pltpu.SemaphoreType.DMA] * _NB,
            [pltpu.SemaphoreType.DMA] * _NB,
        ],
    )
    def body(x_hbm, w_hbm, out_hbm, idx_v, rows_v, gsems, osems):
        wid = lax.axis_index("s") * _NUM_CORES + lax.axis_index("c")
        base = wid * rows_per_w

        def fire_gathers(g, b):
            for j in range(_R):
                for off, ln in splits:
                    pltpu.async_copy(
                        w_hbm.at[idx_v.at[g * _R + j, pl.ds(off, ln)]],
                        rows_v.at[b, j, pl.ds(off, ln)], gsems[b])

        def wait_gathers(b):
            for j in range(_R):
                for off, ln in splits:
                    pltpu.make_async_copy(
                        w_hbm.at[idx_v.at[j, pl.ds(off, ln)]],
                        rows_v.at[b, j, pl.ds(off, ln)], gsems[b]).wait()

        def fire_out(g, b):
            pltpu.async_copy(
                rows_v.at[b], out_hbm.at[pl.ds(base + g * _R, _R)],
                osems[b])

        def wait_out(b):
            pltpu.make_async_copy(
                rows_v.at[b], out_hbm.at[pl.ds(base, _R)], osems[b]).wait()

        def step(g, b, s3, s4):
            wait_gathers(b)
            fire_out(g, b)
            if s3:
                wait_out((b + _NB - 1) % _NB)
            if s4:
                fire_gathers(g + _NB - 1, (b + _NB - 1) % _NB)

        pltpu.sync_copy(x_hbm.at[pl.ds(base, rows_per_w)], idx_v)

        for b in range(_NB - 1):
            fire_gathers(b, b)

        step(0, 0, False, True)
        for g in range(1, _NB):
            step(g, g % _NB, True, True)

        n_steady_outer = (n_chunks - 2 * _NB + 1) // _NB

        def outer(gg, _):
            g0 = gg * _NB
            for b in range(_NB):
                step(g0 + b, b, True, True)
            return ()

        lax.fori_loop(1, 1 + n_steady_outer, outer, (), unroll=False)

        step(n_chunks - 2, (n_chunks - 2) % _NB, True, False)
        step(n_chunks - 1, (n_chunks - 1) % _NB, False, False)

        wait_out((n_chunks - 2) % _NB)
        wait_out((n_chunks - 1) % _NB)

    return body(x, w)


def kernel(x, w):
    return _gather_call(x, w)
